# Initial kernel scaffold; baseline (speedup 1.0000x reference)
import jax
import jax.numpy as jnp
from jax.experimental import pallas as pl


def kernel(x, edge_index, logits, W_in, b_in, Wg, Wl, bl, ln_s, ln_b, W_pred, b_pred):
    raise NotImplementedError("write your pallas kernel here")


# speedup vs baseline: 6.1087x; 6.1087x over previous
import functools

import jax
import jax.numpy as jnp
from jax import lax
from jax.experimental import pallas as pl
from jax.experimental.pallas import tpu as pltpu
from jax.experimental.pallas import tpu_sc as plsc

NC = 2
NS = 16
LANES = 16
CH = 128
DH = 128


def _fill_const(ref, rows, width, value):
    vec = jnp.full((LANES,), value, jnp.float32)

    def body(i, carry):
        for jj in range(width // LANES):
            ref[i, pl.ds(jj * LANES, LANES)] = vec
        return carry

    lax.fori_loop(0, rows, body, 0)


def _zero_shared_rows(zbuf, shared, start, count):
    off = 0
    while off < count:
        rows = min(CH, count - off)
        pltpu.sync_copy(zbuf.at[pl.ds(0, rows)], shared.at[pl.ds(start + off, rows)])
        off += rows


def _split(n):
    rps = (n // NS) // 8 * 8
    tail = n - (NS - 1) * rps
    return rps, tail


def _zero_my_slice(zbuf, shared, s, n):
    rps, tail = _split(n)

    @pl.when(s < NS - 1)
    def _():
        _zero_shared_rows(zbuf, shared, s * rps, rps)

    @pl.when(s == NS - 1)
    def _():
        _zero_shared_rows(zbuf, shared, (NS - 1) * rps, tail + LANES)


def _writeout_my_slice(shared, out_hbm, c, s, n):
    rps, tail = _split(n)

    @pl.when(s < NS - 1)
    def _():
        pltpu.sync_copy(shared.at[pl.ds(s * rps, rps)],
                        out_hbm.at[c, pl.ds(s * rps, rps)])

    @pl.when(s == NS - 1)
    def _():
        pltpu.sync_copy(shared.at[pl.ds((NS - 1) * rps, tail)],
                        out_hbm.at[c, pl.ds((NS - 1) * rps, tail)])


def _sc_degrees(degidx, n):
    epad = degidx.shape[0] // (2 * NS)
    nch = epad // CH
    mesh = plsc.VectorSubcoreMesh(core_axis_name="c", subcore_axis_name="s")

    @functools.partial(
        pl.kernel,
        mesh=mesh,
        out_type=jax.ShapeDtypeStruct((NC, n, DH), jnp.float32),
        scratch_types=[
            pltpu.VMEM((CH,), jnp.int32),
            pltpu.VMEM((CH, DH), jnp.float32),
            pltpu.VMEM_SHARED((n + LANES, DH), jnp.float32),
        ],
    )
    def deg_kernel(idx_hbm, out_hbm, idxv, onesbuf, shared):
        c = lax.axis_index("c")
        s = lax.axis_index("s")
        base = c * (NS * epad) + s * epad
        _fill_const(onesbuf, CH, DH, 0.0)
        _zero_my_slice(onesbuf, shared, s, n)
        _fill_const(onesbuf, CH, DH, 1.0)
        plsc.subcore_barrier()

        def fire(k, carry):
            pltpu.sync_copy(idx_hbm.at[pl.ds(base + k * CH, CH)], idxv)
            pltpu.sync_copy(onesbuf, shared.at[idxv], add=True)
            return carry

        lax.fori_loop(0, nch, fire, 0)
        plsc.subcore_barrier()
        _writeout_my_slice(shared, out_hbm, c, s, n)

    return deg_kernel(degidx)


def _sc_edge(hw2n, gsrc, gdst, n):
    epad = gdst.shape[0] // NS
    nch = epad // CH
    mesh = plsc.VectorSubcoreMesh(core_axis_name="c", subcore_axis_name="s")

    @functools.partial(
        pl.kernel,
        mesh=mesh,
        out_type=jax.ShapeDtypeStruct((NC, n, DH), jnp.float32),
        scratch_types=[
            pltpu.VMEM((CH,), jnp.int32),
            pltpu.VMEM((CH,), jnp.int32),
            pltpu.VMEM((CH,), jnp.int32),
            pltpu.VMEM((CH,), jnp.int32),
            pltpu.VMEM((CH, DH), jnp.float32),
            pltpu.VMEM((CH, DH), jnp.float32),
            pltpu.VMEM_SHARED((n + LANES, DH), jnp.float32),
            pltpu.SemaphoreType.DMA,
            pltpu.SemaphoreType.DMA,
            pltpu.SemaphoreType.DMA,
            pltpu.SemaphoreType.DMA,
        ],
    )
    def edge_kernel(hw_hbm, gsrc_hbm, gdst_hbm, out_hbm,
                    sidx0, sidx1, didx0, didx1, buf0, buf1, shared,
                    g0, g1, s0, s1):
        c = lax.axis_index("c")
        s = lax.axis_index("s")
        sidx = (sidx0, sidx1)
        didx = (didx0, didx1)
        bufs = (buf0, buf1)
        gsems = (g0, g1)
        ssems = (s0, s1)
        sbase = c * (NS * epad) + s * epad
        dbase = s * epad

        _fill_const(buf0, CH, DH, 0.0)
        _zero_my_slice(buf0, shared, s, n)
        plsc.subcore_barrier()

        def pair(j, carry):
            for b in (0, 1):
                k = j * 2 + b

                @pl.when(k >= 2)
                def _():
                    pltpu.make_async_copy(bufs[b], shared.at[didx[b]],
                                          ssems[b]).wait()

                pltpu.sync_copy(gsrc_hbm.at[pl.ds(sbase + k * CH, CH)], sidx[b])
                pltpu.sync_copy(gdst_hbm.at[pl.ds(dbase + k * CH, CH)], didx[b])
                pltpu.async_copy(hw_hbm.at[sidx[b]], bufs[b], gsems[b])
                pltpu.make_async_copy(hw_hbm.at[sidx[b]], bufs[b],
                                      gsems[b]).wait()
                pltpu.async_copy(bufs[b], shared.at[didx[b]], ssems[b],
                                 add=True)
            return carry

        lax.fori_loop(0, nch // 2, pair, 0)
        for b in (0, 1):
            pltpu.make_async_copy(bufs[b], shared.at[didx[b]], ssems[b]).wait()
        plsc.subcore_barrier()
        _writeout_my_slice(shared, out_hbm, c, s, n)

    return edge_kernel(hw2n, gsrc, gdst)


def _rsqrt_deg(deg_col):
    return lax.rsqrt(jnp.maximum(deg_col, 1.0))


def _tc_pre(x, thresh, logits, W_in, b_in, Wg0, degtab, bn):
    n, gene = x.shape
    hid = W_in.shape[1]

    def body(th_ref, lg_ref, x_ref, wi_ref, bi_ref, wg_ref, dg_ref, h_ref, hw_ref):
        mask = (lg_ref[...] >= th_ref[0, 0]).astype(jnp.float32)
        xb = x_ref[...] * mask
        h = jnp.dot(xb, wi_ref[...], preferred_element_type=jnp.float32) + bi_ref[...]
        a = _rsqrt_deg(dg_ref[0, :, 0:1])
        hw = jnp.dot(h, wg_ref[...], preferred_element_type=jnp.float32) * a
        h_ref[...] = h
        hw_ref[0, :, :] = hw[:, :DH]
        hw_ref[1, :, :] = hw[:, DH:]

    return pl.pallas_call(
        body,
        grid=(n // bn,),
        in_specs=[
            pl.BlockSpec((1, 1), lambda i: (0, 0)),
            pl.BlockSpec((1, gene), lambda i: (0, 0)),
            pl.BlockSpec((bn, gene), lambda i: (i, 0)),
            pl.BlockSpec((gene, hid), lambda i: (0, 0)),
            pl.BlockSpec((1, hid), lambda i: (0, 0)),
            pl.BlockSpec((hid, hid), lambda i: (0, 0)),
            pl.BlockSpec((2, bn, DH), lambda i: (0, i, 0)),
        ],
        out_specs=[
            pl.BlockSpec((bn, hid), lambda i: (i, 0)),
            pl.BlockSpec((2, bn, DH), lambda i: (0, i, 0)),
        ],
        out_shape=[
            jax.ShapeDtypeStruct((n, hid), jnp.float32),
            jax.ShapeDtypeStruct((2, n, DH), jnp.float32),
        ],
    )(thresh, logits, x, W_in, b_in, Wg0, degtab)


def _layer_core(h, ag0, ag1, bvec, wl, blv, lns, lnb):
    agg = jnp.concatenate([ag0, ag1], axis=-1) * bvec
    z = agg + jnp.dot(h, wl, preferred_element_type=jnp.float32) + blv
    mu = jnp.mean(z, axis=-1, keepdims=True)
    zc = z - mu
    var = jnp.mean(zc * zc, axis=-1, keepdims=True)
    zn = zc * lax.rsqrt(var + 1e-5) * lns + lnb
    return jnp.maximum(zn, 0.0) + h


def _tc_mid(h, aggr, degtab, Wl_i, bl_i, lns_i, lnb_i, Wg_next, bn):
    n, hid = h.shape

    def body(h_ref, ag_ref, dg_ref, wl_ref, bl_ref, s_ref, b_ref, wg_ref,
             hn_ref, hw_ref):
        bvec = _rsqrt_deg(dg_ref[1, :, 0:1])
        hn = _layer_core(h_ref[...], ag_ref[0, :, :], ag_ref[1, :, :], bvec,
                         wl_ref[...], bl_ref[...], s_ref[...], b_ref[...])
        hn_ref[...] = hn
        a = _rsqrt_deg(dg_ref[0, :, 0:1])
        hw = jnp.dot(hn, wg_ref[...], preferred_element_type=jnp.float32) * a
        hw_ref[0, :, :] = hw[:, :DH]
        hw_ref[1, :, :] = hw[:, DH:]

    return pl.pallas_call(
        body,
        grid=(n // bn,),
        in_specs=[
            pl.BlockSpec((bn, hid), lambda i: (i, 0)),
            pl.BlockSpec((2, bn, DH), lambda i: (0, i, 0)),
            pl.BlockSpec((2, bn, DH), lambda i: (0, i, 0)),
            pl.BlockSpec((hid, hid), lambda i: (0, 0)),
            pl.BlockSpec((1, hid), lambda i: (0, 0)),
            pl.BlockSpec((1, hid), lambda i: (0, 0)),
            pl.BlockSpec((1, hid), lambda i: (0, 0)),
            pl.BlockSpec((hid, hid), lambda i: (0, 0)),
        ],
        out_specs=[
            pl.BlockSpec((bn, hid), lambda i: (i, 0)),
            pl.BlockSpec((2, bn, DH), lambda i: (0, i, 0)),
        ],
        out_shape=[
            jax.ShapeDtypeStruct((n, hid), jnp.float32),
            jax.ShapeDtypeStruct((2, n, DH), jnp.float32),
        ],
    )(h, aggr, degtab, Wl_i, bl_i, lns_i, lnb_i, Wg_next)


def _tc_final(h, aggr, degtab, Wl_i, bl_i, lns_i, lnb_i, W_pred, b_pred, bn):
    n, hid = h.shape
    nout = W_pred.shape[1]

    def body(h_ref, ag_ref, dg_ref, wl_ref, bl_ref, s_ref, b_ref, wp_ref,
             bp_ref, o_ref):
        bvec = _rsqrt_deg(dg_ref[1, :, 0:1])
        hn = _layer_core(h_ref[...], ag_ref[0, :, :], ag_ref[1, :, :], bvec,
                         wl_ref[...], bl_ref[...], s_ref[...], b_ref[...])
        o_ref[...] = jnp.dot(hn, wp_ref[...],
                             preferred_element_type=jnp.float32) + bp_ref[...]

    return pl.pallas_call(
        body,
        grid=(n // bn,),
        in_specs=[
            pl.BlockSpec((bn, hid), lambda i: (i, 0)),
            pl.BlockSpec((2, bn, DH), lambda i: (0, i, 0)),
            pl.BlockSpec((2, bn, DH), lambda i: (0, i, 0)),
            pl.BlockSpec((hid, hid), lambda i: (0, 0)),
            pl.BlockSpec((1, hid), lambda i: (0, 0)),
            pl.BlockSpec((1, hid), lambda i: (0, 0)),
            pl.BlockSpec((1, hid), lambda i: (0, 0)),
            pl.BlockSpec((hid, nout), lambda i: (0, 0)),
            pl.BlockSpec((1, nout), lambda i: (0, 0)),
        ],
        out_specs=pl.BlockSpec((bn, nout), lambda i: (i, 0)),
        out_shape=jax.ShapeDtypeStruct((n, nout), jnp.float32),
    )(h, aggr, degtab, Wl_i, bl_i, lns_i, lnb_i, W_pred, b_pred)


def kernel(x, edge_index, logits, W_in, b_in, Wg, Wl, bl, ln_s, ln_b,
           W_pred, b_pred):
    n, gene = x.shape
    hid = W_in.shape[1]
    nlayers = Wg.shape[0]
    k = 64
    bn = 2000

    thresh = lax.top_k(logits, k)[0][k - 1].reshape(1, 1)
    lg2 = logits.reshape(1, gene)

    e = edge_index.shape[1]
    epw = e // NS
    epad = ((-(-epw // CH) + 1) // 2 * 2) * CH
    src3 = jnp.pad(edge_index[0].reshape(NS, epw), ((0, 0), (0, epad - epw)))
    gsrc = jnp.stack([src3, src3 + n]).reshape(-1)
    gdst = jnp.pad(edge_index[1].reshape(NS, epw), ((0, 0), (0, epad - epw)),
                   constant_values=n).reshape(-1)
    didx = jnp.pad(edge_index.reshape(2, NS, epw),
                   ((0, 0), (0, 0), (0, epad - epw)),
                   constant_values=n).reshape(-1)

    degtab = _sc_degrees(didx, n)
    h, hws = _tc_pre(x, thresh, lg2, W_in, b_in.reshape(1, hid), Wg[0],
                     degtab, bn)
    for i in range(nlayers):
        aggr = _sc_edge(hws.reshape(2 * n, DH), gsrc, gdst, n)
        if i + 1 < nlayers:
            h, hws = _tc_mid(h, aggr, degtab, Wl[i], bl[i].reshape(1, hid),
                             ln_s[i].reshape(1, hid), ln_b[i].reshape(1, hid),
                             Wg[i + 1], bn)
        else:
            out = _tc_final(h, aggr, degtab, Wl[i], bl[i].reshape(1, hid),
                            ln_s[i].reshape(1, hid), ln_b[i].reshape(1, hid),
                            W_pred, b_pred.reshape(1, -1), bn)
    return out
